# baseline (speedup 1.0000x reference)
import jax
import jax.numpy as jnp
import numpy as np
from jax import lax
from jax.experimental import pallas as pl
from jax.experimental.pallas import tpu as pltpu
from jax.experimental.pallas import tpu_sc as plsc

GRID_N = 200
NBINS = GRID_N * GRID_N
NORM = float(25000 * 80)
CLIP_MAX = np.float32(GRID_N - 1 - 1e-6)

NC = 2
NS = 16
L = 16
BLK = 128

TB = 64
T = TB * BLK
UNROLL = 8


def _bin16(xv, yv):
  xv = jnp.clip(xv, 0.0, CLIP_MAX)
  yv = jnp.clip(yv, 0.0, CLIP_MAX)
  xi = (xv + 0.5).astype(jnp.int32)
  yi = (yv + 0.5).astype(jnp.int32)
  return xi * GRID_N + yi


def _make_sc_body(n_tail_valid, tail_pad):
  def _sc_hist_body(z_hbm, tail_hbm, out_hbm,
                    bufa, bufb, tailbuf, hist, sema, semb, semt):
    nw = NC * NS
    nb = z_hbm.shape[0]
    num_tiles = -(-nb // TB)
    tiles_per_worker = -(-num_tiles // nw)
    last_base = nb - TB

    wid = lax.axis_index("c") * NS + lax.axis_index("s")

    iota = lax.iota(jnp.int32, L)
    ones_f = jnp.ones((L,), jnp.float32)

    bufs = (bufa, bufb)
    sems = (sema, semb)

    tail_handle = pltpu.async_copy(tail_hbm, tailbuf, semt)

    def _start(t):
      base = jnp.minimum((wid + t * nw) * TB, last_base)
      s = t % 2
      return pltpu.async_copy(z_hbm.at[pl.ds(base, TB)], bufs[s], sems[s])

    handles = [None, None]
    handles[0] = _start(0)

    zeros_f = jnp.zeros((L,), jnp.float32)

    def _zero(i):
      hist[pl.ds(i * L, L)] = zeros_f
    plsc.parallel_loop(0, NBINS // L, 1, unroll=8)(_zero)

    for t in range(tiles_per_worker):
      b = t % 2
      if t + 1 < tiles_per_worker:
        handles[(t + 1) % 2] = _start(t + 1)
      handles[b].wait()

      tile_start = (wid + t * nw) * T
      base_pts = jnp.minimum(tile_start, last_base * BLK)
      off = tile_start - base_pts
      buf = bufs[b]

      if t + 1 < tiles_per_worker:
        def _group(g, buf=buf):
          blk = g >> 3
          j = (g & 7) * L
          flat = _bin16(buf[blk, 0, pl.ds(j, L)], buf[blk, 1, pl.ds(j, L)])
          plsc.addupdate_scatter(hist, [flat], ones_f)
      else:
        def _group(g, buf=buf, off=off):
          blk = g >> 3
          j = (g & 7) * L
          flat = _bin16(buf[blk, 0, pl.ds(j, L)], buf[blk, 1, pl.ds(j, L)])
          valid = (g * L + iota) >= off
          plsc.addupdate_scatter(hist, [flat], ones_f, mask=valid)

      plsc.parallel_loop(0, T // L, 1, unroll=UNROLL)(_group)

    tail_handle.wait()
    if n_tail_valid:
      @pl.when(wid == 0)
      def _tail():
        def _tgroup(g):
          flat = _bin16(tailbuf[0, pl.ds(g * L, L)],
                        tailbuf[1, pl.ds(g * L, L)])
          valid = (g * L + iota) < n_tail_valid
          plsc.addupdate_scatter(hist, [flat], ones_f, mask=valid)
        plsc.parallel_loop(0, tail_pad // L, 1, unroll=1)(_tgroup)

    pltpu.sync_copy(hist, out_hbm.at[wid])

  return _sc_hist_body


def _merge_body(partials_ref, grid_ref, out_ref):
  s = jnp.sum(partials_ref[...], axis=0)
  prob = s / NORM
  out_ref[...] = jnp.where(grid_ref[...] != 0, 0.0, prob)


def _make_sc_call(n_tail, tail_pad):
  return pl.kernel(
      _make_sc_body(n_tail, tail_pad),
      out_type=jax.ShapeDtypeStruct((NC * NS, NBINS), jnp.float32),
      mesh=plsc.VectorSubcoreMesh(
          core_axis_name="c", subcore_axis_name="s",
          num_cores=NC, num_subcores=NS),
      compiler_params=pltpu.CompilerParams(needs_layout_passes=False),
      scratch_types=[
          pltpu.VMEM((TB, 2, BLK), jnp.float32),
          pltpu.VMEM((TB, 2, BLK), jnp.float32),
          pltpu.VMEM((2, tail_pad), jnp.float32),
          pltpu.VMEM((NBINS,), jnp.float32),
          pltpu.SemaphoreType.DMA,
          pltpu.SemaphoreType.DMA,
          pltpu.SemaphoreType.DMA,
      ],
  )


def _merge2_body(pa_ref, pb_ref, grid_ref, out_ref):
  s = jnp.sum(pa_ref[...], axis=0) + jnp.sum(pb_ref[...], axis=0)
  prob = s / NORM
  out_ref[...] = jnp.where(grid_ref[...] != 0, 0.0, prob)


@jax.jit
def kernel(all_points, grid):
  n = all_points.shape[0]
  nb = n // BLK
  nh = nb * BLK
  n_tail = n - nh
  tail_pad = -(-n_tail // L) * L if n_tail else L

  nba = ((nb // 2) // TB) * TB
  nha = nba * BLK
  nbb = nb - nba

  za = all_points[:nha].reshape(nba, BLK, 2).transpose(0, 2, 1)
  zb = all_points[nha:nh].reshape(nbb, BLK, 2).transpose(0, 2, 1)
  tail = jnp.pad(all_points[nh:].T, ((0, 0), (0, tail_pad - n_tail)))
  dummy_tail = jnp.zeros((2, tail_pad), jnp.float32)

  grid_flat = grid.reshape(-1)
  sc_a = _make_sc_call(0, tail_pad)
  sc_b = _make_sc_call(n_tail, tail_pad)
  pa = sc_a(za, dummy_tail)
  pb = sc_b(zb, tail)
  merge = pl.pallas_call(
      _merge2_body,
      out_shape=jax.ShapeDtypeStruct((NBINS,), jnp.float32),
  )
  return merge(pa, pb, grid_flat).reshape(GRID_N, GRID_N)


# speedup vs baseline: 1.1925x; 1.1925x over previous
import jax
import jax.numpy as jnp
import numpy as np
from jax import lax
from jax.experimental import pallas as pl
from jax.experimental.pallas import tpu as pltpu
from jax.experimental.pallas import tpu_sc as plsc

GRID_N = 200
NBINS = GRID_N * GRID_N
NORM = float(25000 * 80)
CLIP_MAX = np.float32(GRID_N - 1 - 1e-6)

NC = 2
NS = 16
L = 16
BLK = 128

TB = 64
T = TB * BLK
UNROLL = 8


def _bin16(xv, yv):
  xv = jnp.clip(xv, 0.0, CLIP_MAX)
  yv = jnp.clip(yv, 0.0, CLIP_MAX)
  xi = (xv + 0.5).astype(jnp.int32)
  yi = (yv + 0.5).astype(jnp.int32)
  return xi * GRID_N + yi


def _make_sc_body(n_tail_valid, tail_pad):
  def _sc_hist_body(z_hbm, tail_hbm, out_hbm,
                    bufa, bufb, tailbuf, hist, sema, semb, semt):
    nw = NC * NS
    nb = z_hbm.shape[0]
    num_tiles = -(-nb // TB)
    tiles_per_worker = -(-num_tiles // nw)
    last_base = nb - TB

    wid = lax.axis_index("c") * NS + lax.axis_index("s")

    iota = lax.iota(jnp.int32, L)
    ones_f = jnp.ones((L,), jnp.float32)

    bufs = (bufa, bufb)
    sems = (sema, semb)

    tail_handle = pltpu.async_copy(tail_hbm, tailbuf, semt)

    def _start(t):
      base = jnp.minimum((wid + t * nw) * TB, last_base)
      s = t % 2
      return pltpu.async_copy(z_hbm.at[pl.ds(base, TB)], bufs[s], sems[s])

    handles = [None, None]
    handles[0] = _start(0)

    zeros_f = jnp.zeros((L,), jnp.float32)

    def _zero(i):
      hist[pl.ds(i * L, L)] = zeros_f
    plsc.parallel_loop(0, NBINS // L, 1, unroll=8)(_zero)

    for t in range(tiles_per_worker):
      b = t % 2
      if t + 1 < tiles_per_worker:
        handles[(t + 1) % 2] = _start(t + 1)
      handles[b].wait()

      tile_start = (wid + t * nw) * T
      base_pts = jnp.minimum(tile_start, last_base * BLK)
      off = tile_start - base_pts
      buf = bufs[b]

      if t + 1 < tiles_per_worker:
        def _group(g, buf=buf):
          blk = g >> 3
          j = (g & 7) * L
          flat = _bin16(buf[blk, 0, pl.ds(j, L)], buf[blk, 1, pl.ds(j, L)])
          plsc.addupdate_scatter(hist, [flat], ones_f)
      else:
        def _group(g, buf=buf, off=off):
          blk = g >> 3
          j = (g & 7) * L
          flat = _bin16(buf[blk, 0, pl.ds(j, L)], buf[blk, 1, pl.ds(j, L)])
          valid = (g * L + iota) >= off
          plsc.addupdate_scatter(hist, [flat], ones_f, mask=valid)

      plsc.parallel_loop(0, T // L, 1, unroll=UNROLL)(_group)

    tail_handle.wait()
    if n_tail_valid:
      @pl.when(wid == 0)
      def _tail():
        def _tgroup(g):
          flat = _bin16(tailbuf[0, pl.ds(g * L, L)],
                        tailbuf[1, pl.ds(g * L, L)])
          valid = (g * L + iota) < n_tail_valid
          plsc.addupdate_scatter(hist, [flat], ones_f, mask=valid)
        plsc.parallel_loop(0, tail_pad // L, 1, unroll=1)(_tgroup)

    pltpu.sync_copy(hist, out_hbm.at[wid])

  return _sc_hist_body


def _merge_body(partials_ref, grid_ref, out_ref):
  s = jnp.sum(partials_ref[...], axis=0)
  prob = s / NORM
  out_ref[...] = jnp.where(grid_ref[...] != 0, 0.0, prob)


@jax.jit
def kernel(all_points, grid):
  n = all_points.shape[0]
  nb = n // BLK
  nh = nb * BLK
  n_tail = n - nh
  tail_pad = -(-n_tail // L) * L if n_tail else L

  z3 = all_points[:nh].reshape(nb, BLK, 2).transpose(0, 2, 1)
  tail = jnp.pad(all_points[nh:].T, ((0, 0), (0, tail_pad - n_tail)))

  grid_flat = grid.reshape(-1)
  sc_hist = pl.kernel(
      _make_sc_body(n_tail, tail_pad),
      out_type=jax.ShapeDtypeStruct((NC * NS, NBINS), jnp.float32),
      mesh=plsc.VectorSubcoreMesh(
          core_axis_name="c", subcore_axis_name="s",
          num_cores=NC, num_subcores=NS),
      compiler_params=pltpu.CompilerParams(needs_layout_passes=False),
      scratch_types=[
          pltpu.VMEM((TB, 2, BLK), jnp.float32),
          pltpu.VMEM((TB, 2, BLK), jnp.float32),
          pltpu.VMEM((2, tail_pad), jnp.float32),
          pltpu.VMEM((NBINS,), jnp.float32),
          pltpu.SemaphoreType.DMA,
          pltpu.SemaphoreType.DMA,
          pltpu.SemaphoreType.DMA,
      ],
  )
  partials = sc_hist(z3, tail)
  merge = pl.pallas_call(
      _merge_body,
      out_shape=jax.ShapeDtypeStruct((NBINS,), jnp.float32),
  )
  return merge(partials, grid_flat).reshape(GRID_N, GRID_N)
